# Initial kernel scaffold; baseline (speedup 1.0000x reference)
import functools

import jax
import jax.numpy as jnp
from jax import lax
from jax.experimental import pallas as pl
from jax.experimental.pallas import tpu as pltpu
from jax.experimental.pallas import tpu_sc as plsc

N = 10000
NP = 10240
E = 160000
EP = 163840
NT = 32
EPT = EP // NT
NG = EPT // 16
NFEAT = 256
HEADS = 8
NHID = 64
NCLASS = 64
F32 = jnp.float32
I32 = jnp.int32

_LANE = jnp.arange(16, dtype=I32)
_SHIFT8 = jnp.array([8, 9, 10, 11, 12, 13, 14, 15] * 2, dtype=I32)


def _take16(v, idx):
    dn = jax.lax.GatherDimensionNumbers(
        offset_dims=(), collapsed_slice_dims=(0,), start_index_map=(0,))
    return jax.lax.gather(v, idx[:, None], dn, (1,),
                          mode=jax.lax.GatherScatterMode.PROMISE_IN_BOUNDS)


def _dot(a, b):
    return jax.lax.dot_general(a, b, (((1,), (0,)), ((), ())),
                               preferred_element_type=F32)


def _tc1_body(x_ref, w1_ref, a1_ref, xp_ref, ab_ref):
    xp = _dot(x_ref[...], w1_ref[...])
    xp_ref[...] = xp
    ab_ref[...] = _dot(xp, a1_ref[...])


def _tc1(xpad, W1, A1):
    nb = NP // 512
    return pl.pallas_call(
        _tc1_body,
        grid=(nb,),
        in_specs=[
            pl.BlockSpec((512, NFEAT), lambda i: (i, 0)),
            pl.BlockSpec((NFEAT, 512), lambda i: (0, 0)),
            pl.BlockSpec((512, 16), lambda i: (0, 0)),
        ],
        out_specs=[
            pl.BlockSpec((512, 512), lambda i: (i, 0)),
            pl.BlockSpec((512, 16), lambda i: (i, 0)),
        ],
        out_shape=[
            jax.ShapeDtypeStruct((NP, 512), F32),
            jax.ShapeDtypeStruct((NP, 16), F32),
        ],
    )(xpad, W1, A1)


def _tc2_body(a0_ref, a1_ref, xp1_ref, ab1_ref, b1_ref, w2_ref, a2_ref,
              xp2_ref, ab2_ref):
    acc = a0_ref[...] + a1_ref[...]
    ab = ab1_ref[...]
    asl = ab[:, 0:8] + ab[:, 8:16]
    ws = jnp.exp(jnp.maximum(asl, 0.2 * asl))
    xp = xp1_ref[...]
    cols = []
    for h in range(HEADS):
        sl = slice(h * NHID, (h + 1) * NHID)
        num = acc[:, sl] + ws[:, h:h + 1] * xp[:, sl]
        den = acc[:, 512 + h:513 + h] + ws[:, h:h + 1]
        cols.append(num / den)
    hm = jnp.concatenate(cols, axis=1) + b1_ref[...]
    ha = jnp.where(hm > 0, hm, jnp.exp(jnp.minimum(hm, 0.0)) - 1.0)
    xp2 = _dot(ha, w2_ref[...])
    xp2_ref[...] = xp2
    ab2_ref[...] = _dot(xp2, a2_ref[...])


def _tc2(acc0, acc1, xp1, ab1, b1, W2, A2):
    nb = NP // 512
    return pl.pallas_call(
        _tc2_body,
        grid=(nb,),
        in_specs=[
            pl.BlockSpec((512, 528), lambda i: (i, 0)),
            pl.BlockSpec((512, 528), lambda i: (i, 0)),
            pl.BlockSpec((512, 512), lambda i: (i, 0)),
            pl.BlockSpec((512, 16), lambda i: (i, 0)),
            pl.BlockSpec((1, 512), lambda i: (0, 0)),
            pl.BlockSpec((512, NCLASS), lambda i: (0, 0)),
            pl.BlockSpec((NCLASS, 16), lambda i: (0, 0)),
        ],
        out_specs=[
            pl.BlockSpec((512, NCLASS), lambda i: (i, 0)),
            pl.BlockSpec((512, 16), lambda i: (i, 0)),
        ],
        out_shape=[
            jax.ShapeDtypeStruct((NP, NCLASS), F32),
            jax.ShapeDtypeStruct((NP, 16), F32),
        ],
    )(acc0, acc1, xp1, ab1, b1, W2, A2)


def _tc3_body(a0_ref, a1_ref, xp2_ref, ab2_ref, b2_ref, out_ref):
    acc = a0_ref[...] + a1_ref[...]
    ab = ab2_ref[...]
    asl = ab[:, 0:1] + ab[:, 8:9]
    ws = jnp.exp(jnp.maximum(asl, 0.2 * asl))
    num = acc[:, 0:NCLASS] + ws * xp2_ref[...]
    den = acc[:, NCLASS:NCLASS + 1] + ws
    o = num / den + b2_ref[...]
    m = jnp.max(o, axis=1, keepdims=True)
    z = o - m
    lse = jnp.log(jnp.sum(jnp.exp(z), axis=1, keepdims=True))
    out_ref[...] = z - lse


def _tc3(acc0, acc1, xp2, ab2, b2):
    nb = NP // 512
    return pl.pallas_call(
        _tc3_body,
        grid=(nb,),
        in_specs=[
            pl.BlockSpec((512, 80), lambda i: (i, 0)),
            pl.BlockSpec((512, 80), lambda i: (i, 0)),
            pl.BlockSpec((512, NCLASS), lambda i: (i, 0)),
            pl.BlockSpec((512, 16), lambda i: (i, 0)),
            pl.BlockSpec((1, NCLASS), lambda i: (0, 0)),
        ],
        out_specs=pl.BlockSpec((512, NCLASS), lambda i: (i, 0)),
        out_shape=jax.ShapeDtypeStruct((NP, NCLASS), F32),
    )(acc0, acc1, xp2, ab2, b2)


def _make_sc(ch_tot, heads, chunk, nchunk):
    dw = ch_tot + 16
    rows_per_tile = chunk // 16
    nfull = rows_per_tile // 64
    rem = rows_per_tile - nfull * 64
    nq = ch_tot // heads // 16

    mesh = plsc.VectorSubcoreMesh(core_axis_name="c", subcore_axis_name="s",
                                  num_cores=2, num_subcores=16)

    @functools.partial(
        pl.kernel,
        out_type=jax.ShapeDtypeStruct((2, NP, dw), F32),
        mesh=mesh,
        scratch_types=[
            pltpu.VMEM((EPT,), I32),
            pltpu.VMEM((EPT,), I32),
            pltpu.VMEM((EPT,), I32),
            pltpu.VMEM((EPT,), I32),
            pltpu.VMEM((EPT,), I32),
            pltpu.VMEM((64,), I32),
            pltpu.VMEM((64,), I32),
            pltpu.VMEM((64,), I32),
            pltpu.VMEM((64, ch_tot), F32),
            pltpu.VMEM((64, dw), F32),
            pltpu.VMEM((64, 16), F32),
            pltpu.VMEM((64, 16), F32),
            pltpu.SemaphoreType.DMA,
            pltpu.VMEM_SHARED((chunk, dw), F32),
        ],
    )
    def sc(src_hbm, dst_hbm, ab_hbm, xp_hbm, out_hbm,
           src_v, dst_v, csrc, cdl, cgd, src64, dl64, gd64,
           grows, srows, absrc, abdst, sem, accS):
        cid = lax.axis_index("c")
        tid = lax.axis_index("s")
        wid = cid * 16 + tid
        base = wid * EPT
        pltpu.sync_copy(src_hbm.at[pl.ds(base, EPT)], src_v)
        pltpu.sync_copy(dst_hbm.at[pl.ds(base, EPT)], dst_v)

        zero16i = jnp.zeros((16,), I32)

        def init_body(g, carry):
            csrc[pl.ds(g * 16, 16)] = zero16i
            cdl[pl.ds(g * 16, 16)] = zero16i
            cgd[pl.ds(g * 16, 16)] = zero16i
            return carry
        lax.fori_loop(0, NG, init_body, 0)

        my0 = tid * rows_per_tile

        for c in range(nchunk):
            cbase = c * chunk

            def scan_body(g, cnt):
                d16 = dst_v[pl.ds(g * 16, 16)]
                s16 = src_v[pl.ds(g * 16, 16)]
                gid = jnp.full((16,), base + g * 16, I32) + _LANE
                m = (d16 >= cbase) & (d16 < cbase + chunk) & (gid < E)
                mi = m.astype(I32)
                excl = plsc.cumsum(mi) - mi
                tgt = jnp.minimum(jnp.full((16,), cnt, I32) + excl, EPT - 1)
                plsc.store_scatter(csrc, [tgt], s16, mask=m)
                plsc.store_scatter(cdl, [tgt], d16 - cbase, mask=m)
                plsc.store_scatter(cgd, [tgt], d16, mask=m)
                return cnt + jnp.sum(mi)
            cnt = lax.fori_loop(0, NG, scan_body, jnp.int32(0))

            zero16f = jnp.zeros((16,), F32)

            def zs_body(k, carry):
                srows[k // (dw // 16), pl.ds((k % (dw // 16)) * 16, 16)] = \
                    zero16f
                return carry
            lax.fori_loop(0, 64 * (dw // 16), zs_body, 0)
            for j in range(nfull):
                pltpu.sync_copy(srows, accS.at[pl.ds(my0 + j * 64, 64)])
            if rem:
                pltpu.sync_copy(srows.at[pl.ds(0, rem)],
                                accS.at[pl.ds(my0 + nfull * 64, rem)])
            plsc.subcore_barrier()

            def batch_body(b, carry):
                boff = b * 64
                for j in range(4):
                    src64[pl.ds(j * 16, 16)] = csrc[pl.ds(boff + j * 16, 16)]
                    dl64[pl.ds(j * 16, 16)] = cdl[pl.ds(boff + j * 16, 16)]
                    gd64[pl.ds(j * 16, 16)] = cgd[pl.ds(boff + j * 16, 16)]
                cp1 = pltpu.async_copy(xp_hbm.at[src64], grows, sem)
                cp2 = pltpu.async_copy(ab_hbm.at[src64], absrc, sem)
                cp3 = pltpu.async_copy(ab_hbm.at[gd64], abdst, sem)
                cp1.wait()
                cp2.wait()
                cp3.wait()

                def row_body(r, carry2):
                    av = absrc[r, :]
                    bv = abdst[r, :]
                    al = av + _take16(bv, _SHIFT8)
                    w16 = jnp.exp(jnp.maximum(al, 0.2 * al))
                    rg = jnp.full((16,), boff + r, I32)
                    keep = rg < jnp.full((16,), cnt, I32)
                    w16 = jnp.where(keep, w16, 0.0)
                    srows[r, pl.ds(ch_tot, 16)] = jnp.where(
                        _LANE < heads, w16, 0.0)
                    for h in range(heads):
                        wsp = _take16(w16, jnp.full((16,), h, I32))
                        for q in range(nq):
                            col = (h * nq + q) * 16
                            srows[r, pl.ds(col, 16)] = \
                                wsp * grows[r, pl.ds(col, 16)]
                    return carry2
                lax.fori_loop(0, 64, row_body, 0)
                pltpu.sync_copy(srows, accS.at[dl64], add=True)
                return carry
            nb = (cnt + 63) // 64
            lax.fori_loop(0, nb, batch_body, 0)
            plsc.subcore_barrier()

            for j in range(nfull):
                pltpu.sync_copy(
                    accS.at[pl.ds(my0 + j * 64, 64)],
                    out_hbm.at[cid, pl.ds(cbase + my0 + j * 64, 64)])
            if rem:
                pltpu.sync_copy(
                    accS.at[pl.ds(my0 + nfull * 64, rem)],
                    out_hbm.at[cid, pl.ds(cbase + my0 + nfull * 64, rem)])
            plsc.subcore_barrier()

    return sc


_sc1 = _make_sc(512, HEADS, 2560, 4)
_sc2 = _make_sc(NCLASS, 1, NP, 1)


def kernel(x, edge_index, W1, att_src1, att_dst1, b1, W2, att_src2, att_dst2,
           b2):
    xpad = jnp.zeros((NP, NFEAT), F32).at[:N].set(x)
    src = jnp.concatenate(
        [edge_index[0].astype(I32), jnp.zeros((EP - E,), I32)])
    dst = jnp.concatenate(
        [edge_index[1].astype(I32), jnp.zeros((EP - E,), I32)])

    sel = (jnp.arange(512, dtype=I32)[:, None] // NHID
           == jnp.arange(8, dtype=I32)[None, :]).astype(F32)
    A1 = jnp.concatenate([att_src1.reshape(-1)[:, None] * sel,
                          att_dst1.reshape(-1)[:, None] * sel], axis=1)
    A2 = jnp.zeros((NCLASS, 16), F32)
    A2 = A2.at[:, 0].set(att_src2.reshape(-1)).at[:, 8].set(
        att_dst2.reshape(-1))

    xp1, ab1 = _tc1(xpad, W1, A1)
    acc1 = _sc1(src, dst, ab1, xp1)
    xp2, ab2 = _tc2(acc1[0], acc1[1], xp1, ab1, b1.reshape(1, 512), W2, A2)
    acc2 = _sc2(src, dst, ab2, xp2)
    outp = _tc3(acc2[0], acc2[1], xp2, ab2, b2.reshape(1, NCLASS))
    return outp[:N]


# speedup vs baseline: 19.3352x; 19.3352x over previous
import functools

import jax
import jax.numpy as jnp
from jax import lax
from jax.experimental import pallas as pl
from jax.experimental.pallas import tpu as pltpu
from jax.experimental.pallas import tpu_sc as plsc

N = 10000
NP = 10240
E = 160000
EP = 163840
EPT = EP // 32
NG = EPT // 16
NFEAT = 256
HEADS = 8
NHID = 64
NCLASS = 64
F32 = jnp.float32
I32 = jnp.int32


def _take16(v, idx):
    dn = jax.lax.GatherDimensionNumbers(
        offset_dims=(), collapsed_slice_dims=(0,), start_index_map=(0,))
    return jax.lax.gather(v, idx[:, None], dn, (1,),
                          mode=jax.lax.GatherScatterMode.PROMISE_IN_BOUNDS)


def _dot(a, b):
    return jax.lax.dot_general(a, b, (((1,), (0,)), ((), ())),
                               preferred_element_type=F32)


def _tc1_body(x_ref, w1_ref, a1_ref, xp_ref, ab_ref):
    xp = _dot(x_ref[...], w1_ref[...])
    xp_ref[...] = xp
    ab = _dot(xp, a1_ref[...])
    ab_ref[...] = jnp.concatenate([ab, jnp.zeros((512, 112), F32)], axis=1)


def _tc1(xpad, W1, A1):
    return pl.pallas_call(
        _tc1_body,
        grid=(NP // 512,),
        in_specs=[
            pl.BlockSpec((512, NFEAT), lambda i: (i, 0)),
            pl.BlockSpec((NFEAT, 512), lambda i: (0, 0)),
            pl.BlockSpec((512, 16), lambda i: (0, 0)),
        ],
        out_specs=[
            pl.BlockSpec((512, 512), lambda i: (i, 0)),
            pl.BlockSpec((512, 128), lambda i: (i, 0)),
        ],
        out_shape=[
            jax.ShapeDtypeStruct((NP, 512), F32),
            jax.ShapeDtypeStruct((NP, 128), F32),
        ],
    )(xpad, W1, A1)


def _tc2_body(a0_ref, a1_ref, xp1_ref, ab1_ref, b1_ref, w2_ref, a2_ref,
              out_ref):
    acc = a0_ref[...] + a1_ref[...]
    ab = ab1_ref[...]
    asl = ab[:, 0:8] + ab[:, 8:16]
    ws = jnp.exp(jnp.maximum(asl, 0.2 * asl))
    xp = xp1_ref[...]
    cols = []
    for h in range(HEADS):
        sl = slice(h * NHID, (h + 1) * NHID)
        num = acc[:, sl] + ws[:, h:h + 1] * xp[:, sl]
        den = acc[:, 512 + h:513 + h] + ws[:, h:h + 1]
        cols.append(num / den)
    hm = jnp.concatenate(cols, axis=1) + b1_ref[...]
    ha = jnp.where(hm > 0, hm, jnp.exp(jnp.minimum(hm, 0.0)) - 1.0)
    xp2 = _dot(ha, w2_ref[...])
    ab2 = _dot(xp2, a2_ref[...])
    out_ref[...] = jnp.concatenate(
        [xp2, ab2, jnp.zeros((512, 48), F32)], axis=1)


def _tc2(acc0, acc1, xp1, ab1, b1, W2, A2):
    return pl.pallas_call(
        _tc2_body,
        grid=(NP // 512,),
        in_specs=[
            pl.BlockSpec((512, 640), lambda i: (i, 0)),
            pl.BlockSpec((512, 640), lambda i: (i, 0)),
            pl.BlockSpec((512, 512), lambda i: (i, 0)),
            pl.BlockSpec((512, 128), lambda i: (i, 0)),
            pl.BlockSpec((1, 512), lambda i: (0, 0)),
            pl.BlockSpec((512, NCLASS), lambda i: (0, 0)),
            pl.BlockSpec((NCLASS, 16), lambda i: (0, 0)),
        ],
        out_specs=pl.BlockSpec((512, 128), lambda i: (i, 0)),
        out_shape=jax.ShapeDtypeStruct((NP, 128), F32),
    )(acc0, acc1, xp1, ab1, b1, W2, A2)


def _tc3_body(a0_ref, a1_ref, xw_ref, b2_ref, out_ref):
    acc = a0_ref[...] + a1_ref[...]
    xw = xw_ref[...]
    asl = xw[:, 64:65] + xw[:, 72:73]
    ws = jnp.exp(jnp.maximum(asl, 0.2 * asl))
    num = acc[:, 0:NCLASS] + ws * xw[:, 0:NCLASS]
    den = acc[:, NCLASS:NCLASS + 1] + ws
    o = num / den + b2_ref[...]
    m = jnp.max(o, axis=1, keepdims=True)
    z = o - m
    lse = jnp.log(jnp.sum(jnp.exp(z), axis=1, keepdims=True))
    out_ref[...] = z - lse


def _tc3(acc0, acc1, xw2, b2):
    return pl.pallas_call(
        _tc3_body,
        grid=(NP // 512,),
        in_specs=[
            pl.BlockSpec((512, 128), lambda i: (i, 0)),
            pl.BlockSpec((512, 128), lambda i: (i, 0)),
            pl.BlockSpec((512, 128), lambda i: (i, 0)),
            pl.BlockSpec((1, NCLASS), lambda i: (0, 0)),
        ],
        out_specs=pl.BlockSpec((512, NCLASS), lambda i: (i, 0)),
        out_shape=jax.ShapeDtypeStruct((NP, NCLASS), F32),
    )(acc0, acc1, xw2, b2)


_MESH = plsc.VectorSubcoreMesh(core_axis_name="c", subcore_axis_name="s",
                               num_cores=2, num_subcores=16)
_CP = pltpu.CompilerParams(needs_layout_passes=False)


def _edge_setup(body_extras):
    pass


def _make_sc1(chunk, nchunk):
    BE = 16
    BR = BE * 4
    rpt = chunk * 5 // 16
    nfull = rpt // BR
    assert rpt % BR == 0 and chunk * nchunk == NP

    @functools.partial(
        pl.kernel,
        out_type=jax.ShapeDtypeStruct((2, NP * 5, 128), F32),
        mesh=_MESH,
        compiler_params=_CP,
        scratch_types=[
            pltpu.VMEM((EPT,), I32),
            pltpu.VMEM((EPT,), I32),
            pltpu.VMEM((EPT,), I32),
            pltpu.VMEM((EPT,), I32),
            pltpu.VMEM((BR,), I32),
            pltpu.VMEM((BR,), I32),
            pltpu.VMEM((16,), I32),
            pltpu.VMEM((16,), I32),
            pltpu.VMEM((16,), I32),
            pltpu.VMEM((BR, 128), F32),
            pltpu.VMEM((16, 128), F32),
            pltpu.VMEM((16, 128), F32),
            pltpu.VMEM((16, 128), F32),
            pltpu.SemaphoreType.DMA,
            pltpu.VMEM_SHARED((chunk * 5, 128), F32),
        ],
    )
    def sc(src_hbm, dst_hbm, ab_hbm, xp_hbm, out_hbm,
           src_v, dst_v, csrc, cdl, srcx, dlx, s16b, gd16b, dn16b,
           grows, absrc, abdst, denb, sem, accS):
        cid = lax.axis_index("c")
        tid = lax.axis_index("s")
        wid = cid * 16 + tid
        base = wid * EPT
        pltpu.sync_copy(src_hbm.at[pl.ds(base, EPT)], src_v)
        pltpu.sync_copy(dst_hbm.at[pl.ds(base, EPT)], dst_v)

        lane = lax.broadcasted_iota(I32, (16,), 0)
        zero16i = lane * 0
        one16i = zero16i + 1
        zero16f = plsc.bitcast(zero16i, F32)
        shift8 = (zero16i + 8) + (lane & 7)

        def init_body(g, carry):
            csrc[pl.ds(g * 16, 16)] = zero16i
            cdl[pl.ds(g * 16, 16)] = zero16i
            return carry
        lax.fori_loop(0, NG, init_body, 0)

        def initd_body(k, carry):
            denb[k // 8, pl.ds((k % 8) * 16, 16)] = zero16f
            return carry
        lax.fori_loop(0, 16 * 8, initd_body, 0)

        my0 = tid * rpt

        for c in range(nchunk):
            cbase = c * chunk

            def scan_body(g, cntv):
                d16 = dst_v[pl.ds(g * 16, 16)]
                s16 = src_v[pl.ds(g * 16, 16)]
                gid = jnp.full((16,), base + g * 16, I32) + lane
                m = ((d16 >= (zero16i + cbase))
                     & (d16 < (zero16i + (cbase + chunk)))
                     & (gid < (zero16i + E)))
                mi = jnp.where(m, one16i, zero16i)
                incl = mi
                for dstep in (1, 2, 4, 8):
                    sh = _take16(incl, jnp.maximum(lane - dstep, zero16i))
                    incl = incl + jnp.where(lane >= (zero16i + dstep),
                                            sh, zero16i)
                excl = incl - mi
                tgt = jnp.minimum(cntv + excl, zero16i + (EPT - 1))
                plsc.store_scatter(csrc, [tgt], s16, mask=m)
                plsc.store_scatter(cdl, [tgt], d16 - (zero16i + cbase),
                                   mask=m)
                return cntv + _take16(incl, zero16i + 15)
            cntv = lax.fori_loop(0, NG, scan_body, zero16i)
            cnt = cntv[0]

            def zg_body(k, carry):
                grows[k // 8, pl.ds((k % 8) * 16, 16)] = zero16f
                return carry
            lax.fori_loop(0, BR * 8, zg_body, 0)
            for j in range(nfull):
                pltpu.sync_copy(grows, accS.at[pl.ds(my0 + j * BR, BR)])
            plsc.subcore_barrier()

            def batch_body(b, carry):
                boff = b * BE

                @pl.when(boff < cnt)
                def _do():
                    s16 = csrc[pl.ds(boff, 16)]
                    dl16 = cdl[pl.ds(boff, 16)]
                    s16b[pl.ds(0, 16)] = s16
                    gd16b[pl.ds(0, 16)] = dl16 + (zero16i + cbase)
                    dn16b[pl.ds(0, 16)] = dl16 * 5 + 4
                    for b4 in range(4):
                        kv = zero16i + (16 * b4) + lane
                        rep = kv // 4
                        jv = kv & 3
                        srcx[pl.ds(16 * b4, 16)] = _take16(s16, rep) * 4 + jv
                        dlx[pl.ds(16 * b4, 16)] = _take16(dl16, rep) * 5 + jv
                    cp1 = pltpu.async_copy(xp_hbm.at[srcx], grows, sem)
                    cp2 = pltpu.async_copy(ab_hbm.at[s16b], absrc, sem)
                    cp3 = pltpu.async_copy(ab_hbm.at[gd16b], abdst, sem)
                    cp1.wait()
                    cp2.wait()
                    cp3.wait()

                    def row_body(r, carry2):
                        av = absrc[r, pl.ds(0, 16)]
                        bv = abdst[r, pl.ds(0, 16)]
                        al = av + _take16(bv, shift8)
                        w16 = jnp.exp(jnp.maximum(al, 0.2 * al))
                        keep = jnp.full((16,), boff + r, I32) < cntv
                        w16 = jnp.where(keep, w16, zero16f)
                        denb[r, pl.ds(0, 16)] = jnp.where(
                            lane < (zero16i + HEADS), w16, zero16f)
                        for h in range(HEADS):
                            wsp = _take16(w16, zero16i + h)
                            for q in range(4):
                                col = h * 64 + q * 16
                                rr = r * 4 + col // 128
                                cc = col % 128
                                grows[rr, pl.ds(cc, 16)] = \
                                    wsp * grows[rr, pl.ds(cc, 16)]
                        return carry2
                    lax.fori_loop(0, BE, row_body, 0)
                    pltpu.sync_copy(grows, accS.at[dlx], add=True)
                    pltpu.sync_copy(denb, accS.at[dn16b], add=True)
                return carry
            lax.fori_loop(0, EPT // BE, batch_body, 0)
            plsc.subcore_barrier()

            for j in range(nfull):
                pltpu.sync_copy(
                    accS.at[pl.ds(my0 + j * BR, BR)],
                    out_hbm.at[cid, pl.ds(cbase * 5 + my0 + j * BR, BR)])
            plsc.subcore_barrier()

    return sc


def _make_sc2():
    chunk = NP
    rpt = chunk // 16
    nfull = rpt // 64
    assert rpt % 64 == 0

    @functools.partial(
        pl.kernel,
        out_type=jax.ShapeDtypeStruct((2, NP, 128), F32),
        mesh=_MESH,
        compiler_params=_CP,
        scratch_types=[
            pltpu.VMEM((EPT,), I32),
            pltpu.VMEM((EPT,), I32),
            pltpu.VMEM((EPT,), I32),
            pltpu.VMEM((EPT,), I32),
            pltpu.VMEM((64,), I32),
            pltpu.VMEM((64,), I32),
            pltpu.VMEM((64,), I32),
            pltpu.VMEM((64, 128), F32),
            pltpu.VMEM((64, 128), F32),
            pltpu.SemaphoreType.DMA,
            pltpu.VMEM_SHARED((chunk, 128), F32),
        ],
    )
    def sc(src_hbm, dst_hbm, xp_hbm, out_hbm,
           src_v, dst_v, csrc, cdl, src64, dl64, gd64,
           grows, abdst, sem, accS):
        cid = lax.axis_index("c")
        tid = lax.axis_index("s")
        wid = cid * 16 + tid
        base = wid * EPT
        pltpu.sync_copy(src_hbm.at[pl.ds(base, EPT)], src_v)
        pltpu.sync_copy(dst_hbm.at[pl.ds(base, EPT)], dst_v)

        lane = lax.broadcasted_iota(I32, (16,), 0)
        zero16i = lane * 0
        one16i = zero16i + 1
        zero16f = plsc.bitcast(zero16i, F32)
        shift8 = (zero16i + 8) + (lane & 7)

        def init_body(g, carry):
            csrc[pl.ds(g * 16, 16)] = zero16i
            cdl[pl.ds(g * 16, 16)] = zero16i
            return carry
        lax.fori_loop(0, NG, init_body, 0)

        my0 = tid * rpt

        def scan_body(g, cntv):
            d16 = dst_v[pl.ds(g * 16, 16)]
            s16 = src_v[pl.ds(g * 16, 16)]
            gid = jnp.full((16,), base + g * 16, I32) + lane
            m = gid < (zero16i + E)
            mi = jnp.where(m, one16i, zero16i)
            incl = mi
            for dstep in (1, 2, 4, 8):
                sh = _take16(incl, jnp.maximum(lane - dstep, zero16i))
                incl = incl + jnp.where(lane >= (zero16i + dstep),
                                        sh, zero16i)
            excl = incl - mi
            tgt = jnp.minimum(cntv + excl, zero16i + (EPT - 1))
            plsc.store_scatter(csrc, [tgt], s16, mask=m)
            plsc.store_scatter(cdl, [tgt], d16, mask=m)
            return cntv + _take16(incl, zero16i + 15)
        cntv = lax.fori_loop(0, NG, scan_body, zero16i)
        cnt = cntv[0]

        def zg_body(k, carry):
            grows[k // 8, pl.ds((k % 8) * 16, 16)] = zero16f
            return carry
        lax.fori_loop(0, 64 * 8, zg_body, 0)
        for j in range(nfull):
            pltpu.sync_copy(grows, accS.at[pl.ds(my0 + j * 64, 64)])
        plsc.subcore_barrier()

        def batch_body(b, carry):
            boff = b * 64

            @pl.when(boff < cnt)
            def _do():
                for j in range(4):
                    src64[pl.ds(j * 16, 16)] = csrc[pl.ds(boff + j * 16, 16)]
                    dl16 = cdl[pl.ds(boff + j * 16, 16)]
                    dl64[pl.ds(j * 16, 16)] = dl16
                    gd64[pl.ds(j * 16, 16)] = dl16
                cp1 = pltpu.async_copy(xp_hbm.at[src64], grows, sem)
                cp2 = pltpu.async_copy(xp_hbm.at[gd64], abdst, sem)
                cp1.wait()
                cp2.wait()

                def row_body(r, carry2):
                    av = grows[r, pl.ds(64, 16)]
                    bv = abdst[r, pl.ds(64, 16)]
                    al = av + _take16(bv, shift8)
                    w16 = jnp.exp(jnp.maximum(al, 0.2 * al))
                    keep = jnp.full((16,), boff + r, I32) < cntv
                    w16 = jnp.where(keep, w16, zero16f)
                    grows[r, pl.ds(64, 16)] = jnp.where(
                        lane < one16i, w16, zero16f)
                    wsp = _take16(w16, zero16i)
                    for q in range(4):
                        grows[r, pl.ds(q * 16, 16)] = \
                            wsp * grows[r, pl.ds(q * 16, 16)]
                    return carry2
                lax.fori_loop(0, 64, row_body, 0)
                pltpu.sync_copy(grows, accS.at[dl64], add=True)
            return carry
        lax.fori_loop(0, EPT // 64, batch_body, 0)
        plsc.subcore_barrier()

        for j in range(nfull):
            pltpu.sync_copy(
                accS.at[pl.ds(my0 + j * 64, 64)],
                out_hbm.at[cid, pl.ds(my0 + j * 64, 64)])
        plsc.subcore_barrier()

    return sc


_sc1 = _make_sc1(2048, 5)
_sc2 = _make_sc2()


def kernel(x, edge_index, W1, att_src1, att_dst1, b1, W2, att_src2, att_dst2,
           b2):
    xpad = jnp.zeros((NP, NFEAT), F32).at[:N].set(x)
    src = jnp.concatenate(
        [edge_index[0].astype(I32), jnp.zeros((EP - E,), I32)])
    dst = jnp.concatenate(
        [edge_index[1].astype(I32), jnp.zeros((EP - E,), I32)])

    sel = (jnp.arange(512, dtype=I32)[:, None] // NHID
           == jnp.arange(8, dtype=I32)[None, :]).astype(F32)
    A1 = jnp.concatenate([att_src1.reshape(-1)[:, None] * sel,
                          att_dst1.reshape(-1)[:, None] * sel], axis=1)
    A2 = jnp.zeros((NCLASS, 16), F32)
    A2 = A2.at[:, 0].set(att_src2.reshape(-1)).at[:, 8].set(
        att_dst2.reshape(-1))

    xp1, ab1 = _tc1(xpad, W1, A1)
    acc1 = _sc1(src, dst, ab1, xp1.reshape(NP * 4, 128))
    a1v = acc1.reshape(2, NP, 640)
    xw2 = _tc2(a1v[0], a1v[1], xp1, ab1, b1.reshape(1, 512), W2, A2)
    acc2 = _sc2(src, dst, xw2)
    outp = _tc3(acc2[0], acc2[1], xw2, b2.reshape(1, NCLASS))
    return outp[:N]


# speedup vs baseline: 19.4486x; 1.0059x over previous
import functools

import jax
import jax.numpy as jnp
from jax import lax
from jax.experimental import pallas as pl
from jax.experimental.pallas import tpu as pltpu
from jax.experimental.pallas import tpu_sc as plsc

N = 10000
NP = 10240
E = 160000
EP = 163840
EPT = EP // 32
NG = EPT // 16
NFEAT = 256
HEADS = 8
NHID = 64
NCLASS = 64
F32 = jnp.float32
I32 = jnp.int32


def _take16(v, idx):
    dn = jax.lax.GatherDimensionNumbers(
        offset_dims=(), collapsed_slice_dims=(0,), start_index_map=(0,))
    return jax.lax.gather(v, idx[:, None], dn, (1,),
                          mode=jax.lax.GatherScatterMode.PROMISE_IN_BOUNDS)


def _dot(a, b):
    return jax.lax.dot_general(a, b, (((1,), (0,)), ((), ())),
                               preferred_element_type=F32)


def _tc1_body(x_ref, w1_ref, a1_ref, xp_ref, ab_ref):
    xp = _dot(x_ref[...], w1_ref[...])
    xp_ref[...] = xp
    ab = _dot(xp, a1_ref[...])
    ab_ref[...] = jnp.concatenate([ab, jnp.zeros((512, 112), F32)], axis=1)


def _tc1(xpad, W1, A1):
    return pl.pallas_call(
        _tc1_body,
        grid=(NP // 512,),
        in_specs=[
            pl.BlockSpec((512, NFEAT), lambda i: (i, 0)),
            pl.BlockSpec((NFEAT, 512), lambda i: (0, 0)),
            pl.BlockSpec((512, 16), lambda i: (0, 0)),
        ],
        out_specs=[
            pl.BlockSpec((512, 512), lambda i: (i, 0)),
            pl.BlockSpec((512, 128), lambda i: (i, 0)),
        ],
        out_shape=[
            jax.ShapeDtypeStruct((NP, 512), F32),
            jax.ShapeDtypeStruct((NP, 128), F32),
        ],
    )(xpad, W1, A1)


def _tc2_body(a0_ref, a1_ref, xp1_ref, ab1_ref, b1_ref, w2_ref, a2_ref,
              out_ref):
    acc = a0_ref[...] + a1_ref[...]
    ab = ab1_ref[...]
    asl = ab[:, 0:8] + ab[:, 8:16]
    ws = jnp.exp(jnp.maximum(asl, 0.2 * asl))
    xp = xp1_ref[...]
    cols = []
    for h in range(HEADS):
        sl = slice(h * NHID, (h + 1) * NHID)
        num = acc[:, sl] + ws[:, h:h + 1] * xp[:, sl]
        den = acc[:, 512 + h:513 + h] + ws[:, h:h + 1]
        cols.append(num / den)
    hm = jnp.concatenate(cols, axis=1) + b1_ref[...]
    ha = jnp.where(hm > 0, hm, jnp.exp(jnp.minimum(hm, 0.0)) - 1.0)
    xp2 = _dot(ha, w2_ref[...])
    ab2 = _dot(xp2, a2_ref[...])
    out_ref[...] = jnp.concatenate(
        [xp2, ab2, jnp.zeros((512, 48), F32)], axis=1)


def _tc2(acc0, acc1, xp1, ab1, b1, W2, A2):
    return pl.pallas_call(
        _tc2_body,
        grid=(NP // 512,),
        in_specs=[
            pl.BlockSpec((512, 640), lambda i: (i, 0)),
            pl.BlockSpec((512, 640), lambda i: (i, 0)),
            pl.BlockSpec((512, 512), lambda i: (i, 0)),
            pl.BlockSpec((512, 128), lambda i: (i, 0)),
            pl.BlockSpec((1, 512), lambda i: (0, 0)),
            pl.BlockSpec((512, NCLASS), lambda i: (0, 0)),
            pl.BlockSpec((NCLASS, 16), lambda i: (0, 0)),
        ],
        out_specs=pl.BlockSpec((512, 128), lambda i: (i, 0)),
        out_shape=jax.ShapeDtypeStruct((NP, 128), F32),
    )(acc0, acc1, xp1, ab1, b1, W2, A2)


def _tc3_body(a0_ref, a1_ref, xw_ref, b2_ref, out_ref):
    acc = a0_ref[...] + a1_ref[...]
    xw = xw_ref[...]
    asl = xw[:, 64:65] + xw[:, 72:73]
    ws = jnp.exp(jnp.maximum(asl, 0.2 * asl))
    num = acc[:, 0:NCLASS] + ws * xw[:, 0:NCLASS]
    den = acc[:, NCLASS:NCLASS + 1] + ws
    o = num / den + b2_ref[...]
    m = jnp.max(o, axis=1, keepdims=True)
    z = o - m
    lse = jnp.log(jnp.sum(jnp.exp(z), axis=1, keepdims=True))
    out_ref[...] = z - lse


def _tc3(acc0, acc1, xw2, b2):
    return pl.pallas_call(
        _tc3_body,
        grid=(NP // 512,),
        in_specs=[
            pl.BlockSpec((512, 128), lambda i: (i, 0)),
            pl.BlockSpec((512, 128), lambda i: (i, 0)),
            pl.BlockSpec((512, 128), lambda i: (i, 0)),
            pl.BlockSpec((1, NCLASS), lambda i: (0, 0)),
        ],
        out_specs=pl.BlockSpec((512, NCLASS), lambda i: (i, 0)),
        out_shape=jax.ShapeDtypeStruct((NP, NCLASS), F32),
    )(acc0, acc1, xw2, b2)


_MESH = plsc.VectorSubcoreMesh(core_axis_name="c", subcore_axis_name="s",
                               num_cores=2, num_subcores=16)
_CP = pltpu.CompilerParams(needs_layout_passes=False)


def _edge_setup(body_extras):
    pass


def _make_sc1(chunk, nchunk):
    BE = 32
    BR = BE * 4
    rpt = chunk * 5 // 16
    nfull = rpt // BR
    rem = rpt - nfull * BR
    assert chunk * nchunk == NP

    @functools.partial(
        pl.kernel,
        out_type=jax.ShapeDtypeStruct((2, NP * 5, 128), F32),
        mesh=_MESH,
        compiler_params=_CP,
        scratch_types=[
            pltpu.VMEM((EPT,), I32),
            pltpu.VMEM((EPT,), I32),
            pltpu.VMEM((EPT,), I32),
            pltpu.VMEM((EPT,), I32),
            pltpu.VMEM((BR,), I32),
            pltpu.VMEM((BR,), I32),
            pltpu.VMEM((BE,), I32),
            pltpu.VMEM((BE,), I32),
            pltpu.VMEM((BE,), I32),
            pltpu.VMEM((BR, 128), F32),
            pltpu.VMEM((BE, 128), F32),
            pltpu.VMEM((BE, 128), F32),
            pltpu.VMEM((BE, 128), F32),
            pltpu.SemaphoreType.DMA,
            pltpu.VMEM_SHARED((chunk * 5, 128), F32),
        ],
    )
    def sc(src_hbm, dst_hbm, ab_hbm, xp_hbm, out_hbm,
           src_v, dst_v, csrc, cdl, srcx, dlx, s16b, gd16b, dn16b,
           grows, absrc, abdst, denb, sem, accS):
        cid = lax.axis_index("c")
        tid = lax.axis_index("s")
        wid = cid * 16 + tid
        base = wid * EPT
        pltpu.sync_copy(src_hbm.at[pl.ds(base, EPT)], src_v)
        pltpu.sync_copy(dst_hbm.at[pl.ds(base, EPT)], dst_v)

        lane = lax.broadcasted_iota(I32, (16,), 0)
        zero16i = lane * 0
        one16i = zero16i + 1
        zero16f = plsc.bitcast(zero16i, F32)
        shift8 = (zero16i + 8) + (lane & 7)

        def init_body(g, carry):
            csrc[pl.ds(g * 16, 16)] = zero16i
            cdl[pl.ds(g * 16, 16)] = zero16i
            return carry
        lax.fori_loop(0, NG, init_body, 0)

        def initd_body(k, carry):
            denb[k // 8, pl.ds((k % 8) * 16, 16)] = zero16f
            return carry
        lax.fori_loop(0, BE * 8, initd_body, 0)

        my0 = tid * rpt

        for c in range(nchunk):
            cbase = c * chunk

            def scan_body(g, cntv):
                d16 = dst_v[pl.ds(g * 16, 16)]
                s16 = src_v[pl.ds(g * 16, 16)]
                gid = jnp.full((16,), base + g * 16, I32) + lane
                m = ((d16 >= (zero16i + cbase))
                     & (d16 < (zero16i + (cbase + chunk)))
                     & (gid < (zero16i + E)))
                mi = jnp.where(m, one16i, zero16i)
                incl = mi
                for dstep in (1, 2, 4, 8):
                    sh = _take16(incl, jnp.maximum(lane - dstep, zero16i))
                    incl = incl + jnp.where(lane >= (zero16i + dstep),
                                            sh, zero16i)
                excl = incl - mi
                tgt = jnp.minimum(cntv + excl, zero16i + (EPT - 1))
                plsc.store_scatter(csrc, [tgt], s16, mask=m)
                plsc.store_scatter(cdl, [tgt], d16 - (zero16i + cbase),
                                   mask=m)
                return cntv + _take16(incl, zero16i + 15)
            cntv = lax.fori_loop(0, NG, scan_body, zero16i)
            cnt = cntv[0]

            def zg_body(k, carry):
                grows[k // 8, pl.ds((k % 8) * 16, 16)] = zero16f
                return carry
            lax.fori_loop(0, BR * 8, zg_body, 0)
            for j in range(nfull):
                pltpu.sync_copy(grows, accS.at[pl.ds(my0 + j * BR, BR)])
            if rem:
                pltpu.sync_copy(grows.at[pl.ds(0, rem)],
                                accS.at[pl.ds(my0 + nfull * BR, rem)])
            plsc.subcore_barrier()

            def batch_body(b, carry):
                boff = b * BE

                @pl.when(boff < cnt)
                def _do():
                    for g2 in range(BE // 16):
                        s16 = csrc[pl.ds(boff + g2 * 16, 16)]
                        dl16 = cdl[pl.ds(boff + g2 * 16, 16)]
                        s16b[pl.ds(g2 * 16, 16)] = s16
                        gd16b[pl.ds(g2 * 16, 16)] = dl16 + (zero16i + cbase)
                        dn16b[pl.ds(g2 * 16, 16)] = dl16 * 5 + 4
                        for b4 in range(4):
                            kv = zero16i + (16 * b4) + lane
                            rep = kv // 4
                            jv = kv & 3
                            srcx[pl.ds(g2 * 64 + 16 * b4, 16)] = \
                                _take16(s16, rep) * 4 + jv
                            dlx[pl.ds(g2 * 64 + 16 * b4, 16)] = \
                                _take16(dl16, rep) * 5 + jv
                    cp1 = pltpu.async_copy(xp_hbm.at[srcx], grows, sem)
                    cp2 = pltpu.async_copy(ab_hbm.at[s16b], absrc, sem)
                    cp3 = pltpu.async_copy(ab_hbm.at[gd16b], abdst, sem)
                    cp1.wait()
                    cp2.wait()
                    cp3.wait()

                    def row_body(r, carry2):
                        av = absrc[r, pl.ds(0, 16)]
                        bv = abdst[r, pl.ds(0, 16)]
                        al = av + _take16(bv, shift8)
                        w16 = jnp.exp(jnp.maximum(al, 0.2 * al))
                        keep = jnp.full((16,), boff + r, I32) < cntv
                        w16 = jnp.where(keep, w16, zero16f)
                        denb[r, pl.ds(0, 16)] = jnp.where(
                            lane < (zero16i + HEADS), w16, zero16f)
                        for h in range(HEADS):
                            wsp = _take16(w16, zero16i + h)
                            for q in range(4):
                                col = h * 64 + q * 16
                                rr = r * 4 + col // 128
                                cc = col % 128
                                grows[rr, pl.ds(cc, 16)] = \
                                    wsp * grows[rr, pl.ds(cc, 16)]
                        return carry2
                    lax.fori_loop(0, BE, row_body, 0)
                    pltpu.sync_copy(grows, accS.at[dlx], add=True)
                    pltpu.sync_copy(denb, accS.at[dn16b], add=True)
                return carry
            lax.fori_loop(0, EPT // BE, batch_body, 0)
            plsc.subcore_barrier()

            for j in range(nfull):
                pltpu.sync_copy(
                    accS.at[pl.ds(my0 + j * BR, BR)],
                    out_hbm.at[cid, pl.ds(cbase * 5 + my0 + j * BR, BR)])
            if rem:
                pltpu.sync_copy(
                    accS.at[pl.ds(my0 + nfull * BR, rem)],
                    out_hbm.at[cid,
                               pl.ds(cbase * 5 + my0 + nfull * BR, rem)])
            plsc.subcore_barrier()

    return sc


def _make_sc2():
    chunk = NP
    rpt = chunk // 16
    nfull = rpt // 64
    assert rpt % 64 == 0

    @functools.partial(
        pl.kernel,
        out_type=jax.ShapeDtypeStruct((2, NP, 128), F32),
        mesh=_MESH,
        compiler_params=_CP,
        scratch_types=[
            pltpu.VMEM((EPT,), I32),
            pltpu.VMEM((EPT,), I32),
            pltpu.VMEM((EPT,), I32),
            pltpu.VMEM((EPT,), I32),
            pltpu.VMEM((64,), I32),
            pltpu.VMEM((64,), I32),
            pltpu.VMEM((64,), I32),
            pltpu.VMEM((64, 128), F32),
            pltpu.VMEM((64, 128), F32),
            pltpu.SemaphoreType.DMA,
            pltpu.VMEM_SHARED((chunk, 128), F32),
        ],
    )
    def sc(src_hbm, dst_hbm, xp_hbm, out_hbm,
           src_v, dst_v, csrc, cdl, src64, dl64, gd64,
           grows, abdst, sem, accS):
        cid = lax.axis_index("c")
        tid = lax.axis_index("s")
        wid = cid * 16 + tid
        base = wid * EPT
        pltpu.sync_copy(src_hbm.at[pl.ds(base, EPT)], src_v)
        pltpu.sync_copy(dst_hbm.at[pl.ds(base, EPT)], dst_v)

        lane = lax.broadcasted_iota(I32, (16,), 0)
        zero16i = lane * 0
        one16i = zero16i + 1
        zero16f = plsc.bitcast(zero16i, F32)
        shift8 = (zero16i + 8) + (lane & 7)

        def init_body(g, carry):
            csrc[pl.ds(g * 16, 16)] = zero16i
            cdl[pl.ds(g * 16, 16)] = zero16i
            return carry
        lax.fori_loop(0, NG, init_body, 0)

        my0 = tid * rpt

        def scan_body(g, cntv):
            d16 = dst_v[pl.ds(g * 16, 16)]
            s16 = src_v[pl.ds(g * 16, 16)]
            gid = jnp.full((16,), base + g * 16, I32) + lane
            m = gid < (zero16i + E)
            mi = jnp.where(m, one16i, zero16i)
            incl = mi
            for dstep in (1, 2, 4, 8):
                sh = _take16(incl, jnp.maximum(lane - dstep, zero16i))
                incl = incl + jnp.where(lane >= (zero16i + dstep),
                                        sh, zero16i)
            excl = incl - mi
            tgt = jnp.minimum(cntv + excl, zero16i + (EPT - 1))
            plsc.store_scatter(csrc, [tgt], s16, mask=m)
            plsc.store_scatter(cdl, [tgt], d16, mask=m)
            return cntv + _take16(incl, zero16i + 15)
        cntv = lax.fori_loop(0, NG, scan_body, zero16i)
        cnt = cntv[0]

        def zg_body(k, carry):
            grows[k // 8, pl.ds((k % 8) * 16, 16)] = zero16f
            return carry
        lax.fori_loop(0, 64 * 8, zg_body, 0)
        for j in range(nfull):
            pltpu.sync_copy(grows, accS.at[pl.ds(my0 + j * 64, 64)])
        plsc.subcore_barrier()

        def batch_body(b, carry):
            boff = b * 64

            @pl.when(boff < cnt)
            def _do():
                for j in range(4):
                    src64[pl.ds(j * 16, 16)] = csrc[pl.ds(boff + j * 16, 16)]
                    dl16 = cdl[pl.ds(boff + j * 16, 16)]
                    dl64[pl.ds(j * 16, 16)] = dl16
                    gd64[pl.ds(j * 16, 16)] = dl16
                cp1 = pltpu.async_copy(xp_hbm.at[src64], grows, sem)
                cp2 = pltpu.async_copy(xp_hbm.at[gd64], abdst, sem)
                cp1.wait()
                cp2.wait()

                def row_body(r, carry2):
                    av = grows[r, pl.ds(64, 16)]
                    bv = abdst[r, pl.ds(64, 16)]
                    al = av + _take16(bv, shift8)
                    w16 = jnp.exp(jnp.maximum(al, 0.2 * al))
                    keep = jnp.full((16,), boff + r, I32) < cntv
                    w16 = jnp.where(keep, w16, zero16f)
                    grows[r, pl.ds(64, 16)] = jnp.where(
                        lane < one16i, w16, zero16f)
                    wsp = _take16(w16, zero16i)
                    for q in range(4):
                        grows[r, pl.ds(q * 16, 16)] = \
                            wsp * grows[r, pl.ds(q * 16, 16)]
                    return carry2
                lax.fori_loop(0, 64, row_body, 0)
                pltpu.sync_copy(grows, accS.at[dl64], add=True)
            return carry
        lax.fori_loop(0, EPT // 64, batch_body, 0)
        plsc.subcore_barrier()

        for j in range(nfull):
            pltpu.sync_copy(
                accS.at[pl.ds(my0 + j * 64, 64)],
                out_hbm.at[cid, pl.ds(my0 + j * 64, 64)])
        plsc.subcore_barrier()

    return sc


_sc1 = _make_sc1(1280, 8)
_sc2 = _make_sc2()


def kernel(x, edge_index, W1, att_src1, att_dst1, b1, W2, att_src2, att_dst2,
           b2):
    xpad = jnp.zeros((NP, NFEAT), F32).at[:N].set(x)
    src = jnp.concatenate(
        [edge_index[0].astype(I32), jnp.zeros((EP - E,), I32)])
    dst = jnp.concatenate(
        [edge_index[1].astype(I32), jnp.zeros((EP - E,), I32)])

    sel = (jnp.arange(512, dtype=I32)[:, None] // NHID
           == jnp.arange(8, dtype=I32)[None, :]).astype(F32)
    A1 = jnp.concatenate([att_src1.reshape(-1)[:, None] * sel,
                          att_dst1.reshape(-1)[:, None] * sel], axis=1)
    A2 = jnp.zeros((NCLASS, 16), F32)
    A2 = A2.at[:, 0].set(att_src2.reshape(-1)).at[:, 8].set(
        att_dst2.reshape(-1))

    xp1, ab1 = _tc1(xpad, W1, A1)
    acc1 = _sc1(src, dst, ab1, xp1.reshape(NP * 4, 128))
    a1v = acc1.reshape(2, NP, 640)
    xw2 = _tc2(a1v[0], a1v[1], xp1, ab1, b1.reshape(1, 512), W2, A2)
    acc2 = _sc2(src, dst, xw2)
    outp = _tc3(acc2[0], acc2[1], xw2, b2.reshape(1, NCLASS))
    return outp[:N]


# speedup vs baseline: 25.2606x; 1.2988x over previous
import functools

import jax
import jax.numpy as jnp
from jax import lax
from jax.experimental import pallas as pl
from jax.experimental.pallas import tpu as pltpu
from jax.experimental.pallas import tpu_sc as plsc

N = 10000
NP = 10240
E = 160000
EP = 163840
EPT = EP // 32
NG = EPT // 16
NFEAT = 256
HEADS = 8
NHID = 64
NCLASS = 64
F32 = jnp.float32
I32 = jnp.int32


def _take16(v, idx):
    dn = jax.lax.GatherDimensionNumbers(
        offset_dims=(), collapsed_slice_dims=(0,), start_index_map=(0,))
    return jax.lax.gather(v, idx[:, None], dn, (1,),
                          mode=jax.lax.GatherScatterMode.PROMISE_IN_BOUNDS)


def _dot(a, b):
    return jax.lax.dot_general(a, b, (((1,), (0,)), ((), ())),
                               preferred_element_type=F32)


def _tc1_body(x_ref, w1_ref, a1_ref, xp_ref, ab_ref):
    xp = _dot(x_ref[...], w1_ref[...])
    xp_ref[...] = xp
    ab = _dot(xp, a1_ref[...])
    ab_ref[...] = jnp.concatenate([ab, jnp.zeros((512, 112), F32)], axis=1)


def _tc1(xpad, W1, A1):
    return pl.pallas_call(
        _tc1_body,
        grid=(NP // 512,),
        in_specs=[
            pl.BlockSpec((512, NFEAT), lambda i: (i, 0)),
            pl.BlockSpec((NFEAT, 512), lambda i: (0, 0)),
            pl.BlockSpec((512, 16), lambda i: (0, 0)),
        ],
        out_specs=[
            pl.BlockSpec((512, 512), lambda i: (i, 0)),
            pl.BlockSpec((512, 128), lambda i: (i, 0)),
        ],
        out_shape=[
            jax.ShapeDtypeStruct((NP, 512), F32),
            jax.ShapeDtypeStruct((NP, 128), F32),
        ],
    )(xpad, W1, A1)


def _tc2_body(a0_ref, a1_ref, d0_ref, d1_ref, xp1_ref, ab1_ref, b1_ref,
              w2_ref, a2_ref, out_ref):
    acc = a0_ref[...] + a1_ref[...]
    dent = d0_ref[...] + d1_ref[...]
    ab = ab1_ref[...]
    asl = ab[:, 0:8] + ab[:, 8:16]
    ws = jnp.exp(jnp.maximum(asl, 0.2 * asl))
    xp = xp1_ref[...]
    cols = []
    for h in range(HEADS):
        sl = slice(h * NHID, (h + 1) * NHID)
        num = acc[:, sl] + ws[:, h:h + 1] * xp[:, sl]
        den = dent[:, h:h + 1] + ws[:, h:h + 1]
        cols.append(num / den)
    hm = jnp.concatenate(cols, axis=1) + b1_ref[...]
    ha = jnp.where(hm > 0, hm, jnp.exp(jnp.minimum(hm, 0.0)) - 1.0)
    xp2 = _dot(ha, w2_ref[...])
    ab2 = _dot(xp2, a2_ref[...])
    out_ref[...] = jnp.concatenate(
        [xp2, ab2, jnp.zeros((512, 48), F32)], axis=1)


def _tc2(acc0, acc1, den0, den1, xp1, ab1, b1, W2, A2):
    return pl.pallas_call(
        _tc2_body,
        grid=(NP // 512,),
        in_specs=[
            pl.BlockSpec((512, 512), lambda i: (i, 0)),
            pl.BlockSpec((512, 512), lambda i: (i, 0)),
            pl.BlockSpec((512, 8), lambda i: (i, 0)),
            pl.BlockSpec((512, 8), lambda i: (i, 0)),
            pl.BlockSpec((512, 512), lambda i: (i, 0)),
            pl.BlockSpec((512, 128), lambda i: (i, 0)),
            pl.BlockSpec((1, 512), lambda i: (0, 0)),
            pl.BlockSpec((512, NCLASS), lambda i: (0, 0)),
            pl.BlockSpec((NCLASS, 16), lambda i: (0, 0)),
        ],
        out_specs=pl.BlockSpec((512, 128), lambda i: (i, 0)),
        out_shape=jax.ShapeDtypeStruct((NP, 128), F32),
    )(acc0, acc1, den0, den1, xp1, ab1, b1, W2, A2)


def _tc3_body(a0_ref, a1_ref, xw_ref, b2_ref, out_ref):
    acc = a0_ref[...] + a1_ref[...]
    xw = xw_ref[...]
    asl = xw[:, 64:65] + xw[:, 72:73]
    ws = jnp.exp(jnp.maximum(asl, 0.2 * asl))
    num = acc[:, 0:NCLASS] + ws * xw[:, 0:NCLASS]
    den = acc[:, NCLASS:NCLASS + 1] + ws
    o = num / den + b2_ref[...]
    m = jnp.max(o, axis=1, keepdims=True)
    z = o - m
    lse = jnp.log(jnp.sum(jnp.exp(z), axis=1, keepdims=True))
    out_ref[...] = z - lse


def _tc3(acc0, acc1, xw2, b2):
    return pl.pallas_call(
        _tc3_body,
        grid=(NP // 512,),
        in_specs=[
            pl.BlockSpec((512, 128), lambda i: (i, 0)),
            pl.BlockSpec((512, 128), lambda i: (i, 0)),
            pl.BlockSpec((512, 128), lambda i: (i, 0)),
            pl.BlockSpec((1, NCLASS), lambda i: (0, 0)),
        ],
        out_specs=pl.BlockSpec((512, NCLASS), lambda i: (i, 0)),
        out_shape=jax.ShapeDtypeStruct((NP, NCLASS), F32),
    )(acc0, acc1, xw2, b2)


_MESH = plsc.VectorSubcoreMesh(core_axis_name="c", subcore_axis_name="s",
                               num_cores=2, num_subcores=16)
_CP = pltpu.CompilerParams(needs_layout_passes=False)


def _edge_setup(body_extras):
    pass


def _make_sc1(chunk, nchunk):
    BE = 32
    BR = BE * 4
    rpt = chunk * 4 // 16
    nfull = rpt // BR
    rem = rpt - nfull * BR
    drows = chunk * HEADS // 128
    drt = drows // 16
    assert chunk * nchunk == NP and drows % 16 == 0

    @functools.partial(
        pl.kernel,
        out_type=(
            jax.ShapeDtypeStruct((2, NP * 4, 128), F32),
            jax.ShapeDtypeStruct((2, NP * HEADS // 128, 128), F32),
        ),
        mesh=_MESH,
        compiler_params=_CP,
        scratch_types=[
            pltpu.VMEM((EPT,), I32),
            pltpu.VMEM((EPT,), I32),
            pltpu.VMEM((EPT,), I32),
            pltpu.VMEM((EPT,), I32),
            pltpu.VMEM((BR,), I32),
            pltpu.VMEM((BR,), I32),
            pltpu.VMEM((BR,), I32),
            pltpu.VMEM((BR,), I32),
            pltpu.VMEM((BE,), I32),
            pltpu.VMEM((BE,), I32),
            pltpu.VMEM((BE,), I32),
            pltpu.VMEM((BE,), I32),
            pltpu.VMEM((BR, 128), F32),
            pltpu.VMEM((BR, 128), F32),
            pltpu.VMEM((BE, 128), F32),
            pltpu.VMEM((BE, 128), F32),
            pltpu.VMEM((BE, 128), F32),
            pltpu.VMEM((BE, 128), F32),
            pltpu.VMEM((drows, 128), F32),
            pltpu.VMEM((drows,), I32),
            pltpu.SemaphoreType.DMA,
            pltpu.SemaphoreType.DMA,
            pltpu.VMEM_SHARED((chunk * 4, 128), F32),
            pltpu.VMEM_SHARED((drows, 128), F32),
        ],
    )
    def sc(src_hbm, dst_hbm, ab_hbm, xp_hbm, out_hbm, den_hbm,
           src_v, dst_v, csrc, cdl, srcx0, srcx1, dlx0, dlx1,
           s16b0, s16b1, gd16b0, gd16b1, grows0, grows1,
           absrc0, absrc1, abdst0, abdst1, denacc, drid,
           semg0, semg1, accS, denS):
        bufs = (
            (srcx0, dlx0, s16b0, gd16b0, grows0, absrc0, abdst0, semg0),
            (srcx1, dlx1, s16b1, gd16b1, grows1, absrc1, abdst1, semg1),
        )
        cid = lax.axis_index("c")
        tid = lax.axis_index("s")
        wid = cid * 16 + tid
        base = wid * EPT
        pltpu.sync_copy(src_hbm.at[pl.ds(base, EPT)], src_v)
        pltpu.sync_copy(dst_hbm.at[pl.ds(base, EPT)], dst_v)

        lane = lax.broadcasted_iota(I32, (16,), 0)
        zero16i = lane * 0
        one16i = zero16i + 1
        zero16f = plsc.bitcast(zero16i, F32)
        shift8 = (zero16i + 8) + (lane & 7)

        def init_body(g, carry):
            csrc[pl.ds(g * 16, 16)] = zero16i
            cdl[pl.ds(g * 16, 16)] = zero16i
            return carry
        lax.fori_loop(0, NG, init_body, 0)

        def initr_body(g, carry):
            drid[pl.ds(g * 16, 16)] = jnp.full((16,), g * 16, I32) + lane
            return carry
        lax.fori_loop(0, drows // 16, initr_body, 0)

        my0 = tid * rpt

        def build_idx(boff, buf, cbase):
            srcx, dlx, s16b, gd16b = buf[0], buf[1], buf[2], buf[3]
            for g2 in range(BE // 16):
                s16 = csrc[pl.ds(boff + g2 * 16, 16)]
                dl16 = cdl[pl.ds(boff + g2 * 16, 16)]
                s16b[pl.ds(g2 * 16, 16)] = s16
                gd16b[pl.ds(g2 * 16, 16)] = dl16 + (zero16i + cbase)
                for b4 in range(4):
                    kv = zero16i + (16 * b4) + lane
                    rep = kv // 4
                    jv = kv & 3
                    srcx[pl.ds(g2 * 64 + 16 * b4, 16)] = \
                        _take16(s16, rep) * 4 + jv
                    dlx[pl.ds(g2 * 64 + 16 * b4, 16)] = \
                        _take16(dl16, rep) * 4 + jv

        def issue(buf):
            pltpu.async_copy(xp_hbm.at[buf[0]], buf[4], buf[7])
            pltpu.async_copy(ab_hbm.at[buf[2]], buf[5], buf[7])
            pltpu.async_copy(ab_hbm.at[buf[3]], buf[6], buf[7])

        def wait(buf):
            cp1 = pltpu.make_async_copy(xp_hbm.at[buf[0]], buf[4], buf[7])
            cp2 = pltpu.make_async_copy(ab_hbm.at[buf[2]], buf[5], buf[7])
            cp3 = pltpu.make_async_copy(ab_hbm.at[buf[3]], buf[6], buf[7])
            cp1.wait()
            cp2.wait()
            cp3.wait()

        for c in range(nchunk):
            cbase = c * chunk

            def scan_body(g, cntv):
                d16 = dst_v[pl.ds(g * 16, 16)]
                s16 = src_v[pl.ds(g * 16, 16)]
                gid = jnp.full((16,), base + g * 16, I32) + lane
                m = ((d16 >= (zero16i + cbase))
                     & (d16 < (zero16i + (cbase + chunk)))
                     & (gid < (zero16i + E)))
                mi = jnp.where(m, one16i, zero16i)
                incl = mi
                for dstep in (1, 2, 4, 8):
                    sh = _take16(incl, jnp.maximum(lane - dstep, zero16i))
                    incl = incl + jnp.where(lane >= (zero16i + dstep),
                                            sh, zero16i)
                excl = incl - mi
                tgt = jnp.minimum(cntv + excl, zero16i + (EPT - 1))
                plsc.store_scatter(csrc, [tgt], s16, mask=m)
                plsc.store_scatter(cdl, [tgt], d16 - (zero16i + cbase),
                                   mask=m)
                return cntv + _take16(incl, zero16i + 15)
            cntv = lax.fori_loop(0, NG, scan_body, zero16i)
            cnt = cntv[0]

            def zg_body(k, carry):
                grows0[k // 8, pl.ds((k % 8) * 16, 16)] = zero16f
                return carry
            lax.fori_loop(0, BR * 8, zg_body, 0)

            def zd_body(k, carry):
                denacc[k // 8, pl.ds((k % 8) * 16, 16)] = zero16f
                return carry
            lax.fori_loop(0, drows * 8, zd_body, 0)

            for j in range(nfull):
                pltpu.sync_copy(grows0, accS.at[pl.ds(my0 + j * BR, BR)])
            if rem:
                pltpu.sync_copy(grows0.at[pl.ds(0, rem)],
                                accS.at[pl.ds(my0 + nfull * BR, rem)])
            pltpu.sync_copy(grows0.at[pl.ds(0, drt)],
                            denS.at[pl.ds(tid * drt, drt)])
            plsc.subcore_barrier()

            @pl.when(cnt > 0)
            def _prime():
                build_idx(0, bufs[0], cbase)
                issue(bufs[0])

            def process(bb, buf, nxt):
                boff = bb * BE

                @pl.when(boff < cnt)
                def _do():
                    wait(buf)

                    @pl.when(boff + BE < cnt)
                    def _pf():
                        build_idx(boff + BE, nxt, cbase)
                        issue(nxt)

                    def row_body(r, carry2):
                        av = buf[5][r, pl.ds(0, 16)]
                        bv = buf[6][r, pl.ds(0, 16)]
                        al = av + _take16(bv, shift8)
                        w16 = jnp.exp(jnp.maximum(al, 0.2 * al))
                        keep = jnp.full((16,), boff + r, I32) < cntv
                        w16 = jnp.where(keep, w16, zero16f)
                        g16 = boff + r - (boff + r) // 16 * 16
                        dl16 = cdl[pl.ds(boff + (r // 16) * 16, 16)]
                        dflat = jnp.minimum(
                            _take16(dl16, jnp.full((16,), g16, I32))
                            * 8 + lane, zero16i + (drows * 128 - 1))
                        plsc.addupdate_scatter(
                            denacc, [dflat // 128, dflat - dflat // 128
                                     * 128],
                            w16, mask=lane < (zero16i + HEADS))
                        gr = buf[4]
                        for h in range(HEADS):
                            wsp = _take16(w16, zero16i + h)
                            for q in range(4):
                                col = h * 64 + q * 16
                                rr = r * 4 + col // 128
                                cc = col % 128
                                gr[rr, pl.ds(cc, 16)] = \
                                    wsp * gr[rr, pl.ds(cc, 16)]
                        return carry2
                    lax.fori_loop(0, BE, row_body, 0)
                    pltpu.sync_copy(buf[4], accS.at[buf[1]], add=True)
                return 0

            def pair_body(i, carry):
                process(2 * i, bufs[0], bufs[1])
                process(2 * i + 1, bufs[1], bufs[0])
                return carry
            lax.fori_loop(0, EPT // BE // 2, pair_body, 0)

            pltpu.sync_copy(denacc, denS.at[drid], add=True)
            plsc.subcore_barrier()

            for j in range(nfull):
                pltpu.sync_copy(
                    accS.at[pl.ds(my0 + j * BR, BR)],
                    out_hbm.at[cid, pl.ds(cbase * 4 + my0 + j * BR, BR)])
            if rem:
                pltpu.sync_copy(
                    accS.at[pl.ds(my0 + nfull * BR, rem)],
                    out_hbm.at[cid,
                               pl.ds(cbase * 4 + my0 + nfull * BR, rem)])
            @pl.when(tid < drows // 8)
            def _wbd():
                pltpu.sync_copy(
                    denS.at[pl.ds(tid * 8, 8)],
                    den_hbm.at[cid, pl.ds(c * drows + tid * 8, 8)])
            plsc.subcore_barrier()

    return sc


def _make_sc2(chunk, nchunk):
    BE = 64
    rpt = chunk // 16
    nfull = rpt // BE
    assert rpt % BE == 0 and chunk * nchunk == NP

    @functools.partial(
        pl.kernel,
        out_type=jax.ShapeDtypeStruct((2, NP, 128), F32),
        mesh=_MESH,
        compiler_params=_CP,
        scratch_types=[
            pltpu.VMEM((EPT,), I32),
            pltpu.VMEM((EPT,), I32),
            pltpu.VMEM((EPT,), I32),
            pltpu.VMEM((EPT,), I32),
            pltpu.VMEM((BE,), I32),
            pltpu.VMEM((BE,), I32),
            pltpu.VMEM((BE,), I32),
            pltpu.VMEM((BE,), I32),
            pltpu.VMEM((BE,), I32),
            pltpu.VMEM((BE,), I32),
            pltpu.VMEM((BE, 128), F32),
            pltpu.VMEM((BE, 128), F32),
            pltpu.VMEM((BE, 128), F32),
            pltpu.VMEM((BE, 128), F32),
            pltpu.SemaphoreType.DMA,
            pltpu.SemaphoreType.DMA,
            pltpu.VMEM_SHARED((chunk, 128), F32),
        ],
    )
    def sc(src_hbm, dst_hbm, xp_hbm, out_hbm,
           src_v, dst_v, csrc, cdl, s0, s1, d0, d1, gdb0, gdb1,
           g0, g1, ab0, ab1, semg0, semg1, accS):
        bufs = ((s0, d0, g0, ab0, semg0, gdb0),
                (s1, d1, g1, ab1, semg1, gdb1))
        cid = lax.axis_index("c")
        tid = lax.axis_index("s")
        wid = cid * 16 + tid
        base = wid * EPT
        pltpu.sync_copy(src_hbm.at[pl.ds(base, EPT)], src_v)
        pltpu.sync_copy(dst_hbm.at[pl.ds(base, EPT)], dst_v)

        lane = lax.broadcasted_iota(I32, (16,), 0)
        zero16i = lane * 0
        one16i = zero16i + 1
        zero16f = plsc.bitcast(zero16i, F32)
        shift8 = (zero16i + 8) + (lane & 7)

        def init_body(g, carry):
            csrc[pl.ds(g * 16, 16)] = zero16i
            cdl[pl.ds(g * 16, 16)] = zero16i
            return carry
        lax.fori_loop(0, NG, init_body, 0)

        my0 = tid * rpt

        def build_idx(boff, buf, cbase):
            for j in range(BE // 16):
                dl16 = cdl[pl.ds(boff + j * 16, 16)]
                buf[0][pl.ds(j * 16, 16)] = csrc[pl.ds(boff + j * 16, 16)]
                buf[1][pl.ds(j * 16, 16)] = dl16
                buf[5][pl.ds(j * 16, 16)] = dl16 + (zero16i + cbase)

        def issue(buf):
            pltpu.async_copy(xp_hbm.at[buf[0]], buf[2], buf[4])
            pltpu.async_copy(xp_hbm.at[buf[5]], buf[3], buf[4])

        def wait(buf):
            pltpu.make_async_copy(xp_hbm.at[buf[0]], buf[2], buf[4]).wait()
            pltpu.make_async_copy(xp_hbm.at[buf[5]], buf[3], buf[4]).wait()

        for c in range(nchunk):
            cbase = c * chunk

            def scan_body(g, cntv):
                d16 = dst_v[pl.ds(g * 16, 16)]
                s16 = src_v[pl.ds(g * 16, 16)]
                gid = jnp.full((16,), base + g * 16, I32) + lane
                m = ((d16 >= (zero16i + cbase))
                     & (d16 < (zero16i + (cbase + chunk)))
                     & (gid < (zero16i + E)))
                mi = jnp.where(m, one16i, zero16i)
                incl = mi
                for dstep in (1, 2, 4, 8):
                    sh = _take16(incl, jnp.maximum(lane - dstep, zero16i))
                    incl = incl + jnp.where(lane >= (zero16i + dstep),
                                            sh, zero16i)
                excl = incl - mi
                tgt = jnp.minimum(cntv + excl, zero16i + (EPT - 1))
                plsc.store_scatter(csrc, [tgt], s16, mask=m)
                plsc.store_scatter(cdl, [tgt], d16 - (zero16i + cbase),
                                   mask=m)
                return cntv + _take16(incl, zero16i + 15)
            cntv = lax.fori_loop(0, NG, scan_body, zero16i)
            cnt = cntv[0]

            def zg_body(k, carry):
                g0[k // 8, pl.ds((k % 8) * 16, 16)] = zero16f
                return carry
            lax.fori_loop(0, BE * 8, zg_body, 0)
            for j in range(nfull):
                pltpu.sync_copy(g0, accS.at[pl.ds(my0 + j * BE, BE)])
            plsc.subcore_barrier()

            @pl.when(cnt > 0)
            def _prime():
                build_idx(0, bufs[0], cbase)
                issue(bufs[0])

            def process(bb, buf, nxt):
                boff = bb * BE

                @pl.when(boff < cnt)
                def _do():
                    wait(buf)

                    @pl.when(boff + BE < cnt)
                    def _pf():
                        build_idx(boff + BE, nxt, cbase)
                        issue(nxt)

                    def row_body(r, carry2):
                        gr = buf[2]
                        av = gr[r, pl.ds(64, 16)]
                        bv = buf[3][r, pl.ds(64, 16)]
                        al = av + _take16(bv, shift8)
                        w16 = jnp.exp(jnp.maximum(al, 0.2 * al))
                        keep = jnp.full((16,), boff + r, I32) < cntv
                        w16 = jnp.where(keep, w16, zero16f)
                        gr[r, pl.ds(64, 16)] = jnp.where(
                            lane < one16i, w16, zero16f)
                        wsp = _take16(w16, zero16i)
                        for q in range(4):
                            gr[r, pl.ds(q * 16, 16)] = \
                                wsp * gr[r, pl.ds(q * 16, 16)]
                        return carry2
                    lax.fori_loop(0, BE, row_body, 0)
                    pltpu.sync_copy(buf[2], accS.at[buf[1]], add=True)
                return 0

            def pair_body(i, carry):
                process(2 * i, bufs[0], bufs[1])
                process(2 * i + 1, bufs[1], bufs[0])
                return carry
            lax.fori_loop(0, EPT // BE // 2, pair_body, 0)
            plsc.subcore_barrier()

            for j in range(nfull):
                pltpu.sync_copy(
                    accS.at[pl.ds(my0 + j * BE, BE)],
                    out_hbm.at[cid, pl.ds(cbase + my0 + j * BE, BE)])
            plsc.subcore_barrier()

    return sc


_sc1 = _make_sc1(1280, 8)
_sc2 = _make_sc2(5120, 2)


def kernel(x, edge_index, W1, att_src1, att_dst1, b1, W2, att_src2, att_dst2,
           b2):
    xpad = jnp.zeros((NP, NFEAT), F32).at[:N].set(x)
    src = jnp.concatenate(
        [edge_index[0].astype(I32), jnp.zeros((EP - E,), I32)])
    dst = jnp.concatenate(
        [edge_index[1].astype(I32), jnp.zeros((EP - E,), I32)])

    sel = (jnp.arange(512, dtype=I32)[:, None] // NHID
           == jnp.arange(8, dtype=I32)[None, :]).astype(F32)
    A1 = jnp.concatenate([att_src1.reshape(-1)[:, None] * sel,
                          att_dst1.reshape(-1)[:, None] * sel], axis=1)
    A2 = jnp.zeros((NCLASS, 16), F32)
    A2 = A2.at[:, 0].set(att_src2.reshape(-1)).at[:, 8].set(
        att_dst2.reshape(-1))

    xp1, ab1 = _tc1(xpad, W1, A1)
    acc1, den1 = _sc1(src, dst, ab1, xp1.reshape(NP * 4, 128))
    a1v = acc1.reshape(2, NP, 512)
    d1v = den1.reshape(2, NP, 8)
    xw2 = _tc2(a1v[0], a1v[1], d1v[0], d1v[1], xp1, ab1,
               b1.reshape(1, 512), W2, A2)
    acc2 = _sc2(src, dst, xw2)
    outp = _tc3(acc2[0], acc2[1], xw2, b2.reshape(1, NCLASS))
    return outp[:N]
